# baseline (speedup 1.0000x reference)
import functools
import jax
import jax.numpy as jnp
from jax import lax
from jax.experimental import pallas as pl
from jax.experimental.pallas import tpu as pltpu
from jax.experimental.pallas import tpu_sc as plsc

_C = 96
_S = 448
_NW = 32


def _sdesc(v):
    return plsc.sort_key_val(v, v, descending=True)[0]


def _rev(v):
    return lax.rev(v, dimensions=(0,))


def _merge16(a, b):
    rb = _rev(b)
    hi = jnp.maximum(a, rb)
    lo = jnp.minimum(a, rb)
    return _sdesc(hi), _sdesc(lo)


def _merge32(a, b):
    a0, a1 = a
    b0, b1 = b
    rb0 = _rev(b1)
    rb1 = _rev(b0)
    h0 = jnp.maximum(a0, rb0)
    h1 = jnp.maximum(a1, rb1)
    l0 = jnp.minimum(a0, rb0)
    l1 = jnp.minimum(a1, rb1)
    u0 = jnp.maximum(h0, h1)
    u1 = jnp.minimum(h0, h1)
    w0 = jnp.maximum(l0, l1)
    w1 = jnp.minimum(l0, l1)
    return _sdesc(u0), _sdesc(u1), _sdesc(w0), _sdesc(w1)


def _select47(q, p):
    q0, q1, q2, q3 = q
    p0, p1 = p
    h2 = jnp.maximum(q2, _rev(p1))
    h3 = jnp.maximum(q3, _rev(p0))
    e0 = jnp.minimum(q0, h2)
    e1 = jnp.minimum(q1, h3)
    f = jnp.maximum(e0, e1)
    return jnp.min(f)


def _chunk_compute(in_v, out_v):
    @plsc.parallel_loop(0, _S, unroll=8)
    def loc_body(s):
        sp = jnp.full((16,), s, jnp.int32)
        ys = []
        for j in range(6):
            cvec = lax.iota(jnp.int32, 16) + 16 * j
            v = plsc.load_gather(in_v, [cvec, sp])
            ys.append(jnp.maximum(v, 0.0))
        ss = [_sdesc(y) for y in ys]
        p1 = _merge16(ss[0], ss[1])
        p2 = _merge16(ss[2], ss[3])
        p3 = _merge16(ss[4], ss[5])
        q = _merge32(p1, p2)
        t = _select47(q, p3)
        for j in range(6):
            cvec = lax.iota(jnp.int32, 16) + 16 * j
            o = jnp.where(ys[j] >= t, ys[j], 0.0)
            plsc.store_scatter(out_v, [cvec, sp], o)


def kernel(x):
    B, C, H, W = x.shape
    assert C == _C
    HW = H * W
    RPB = HW // _S
    assert HW % _S == 0
    total_chunks = B * RPB
    CPW = total_chunks // _NW
    assert total_chunks % _NW == 0
    xv = x.reshape(B * C * RPB, _S)

    mesh = plsc.VectorSubcoreMesh(core_axis_name="c", subcore_axis_name="s", num_cores=2, num_subcores=16)

    @functools.partial(
        pl.kernel,
        out_type=jax.ShapeDtypeStruct((B * C * RPB, _S), jnp.float32),
        mesh=mesh,
        scratch_types=[
            pltpu.VMEM((C,), jnp.int32),
            pltpu.VMEM((C, _S), jnp.float32),
            pltpu.VMEM((C, _S), jnp.float32),
            pltpu.SemaphoreType.DMA,
            pltpu.SemaphoreType.DMA,
        ],
        compiler_params=pltpu.CompilerParams(use_tc_tiling_on_sc=False, needs_layout_passes=False),
    )
    def run(x_hbm, o_hbm, idx_v, in_v, out_v, gsem, ssem):
        wid = lax.axis_index("s") * 2 + lax.axis_index("c")

        def chunk_body(k, carry):
            g = wid * CPW + k
            b = g // RPB
            m = g % RPB
            for j in range(6):
                cvec = lax.iota(jnp.int32, 16) + 16 * j
                idx_v[pl.ds(16 * j, 16)] = (b * C + cvec) * RPB + m
            pltpu.async_copy(x_hbm.at[idx_v], in_v, gsem).wait()
            _chunk_compute(in_v, out_v)
            pltpu.async_copy(out_v, o_hbm.at[idx_v], ssem).wait()
            return carry

        lax.fori_loop(0, CPW, chunk_body, 0)

    out = run(xv)
    return out.reshape(B, C, H, W)


# speedup vs baseline: 1.1918x; 1.1918x over previous
import functools
import jax
import jax.numpy as jnp
from jax import lax
from jax.experimental import pallas as pl
from jax.experimental.pallas import tpu as pltpu
from jax.experimental.pallas import tpu_sc as plsc

_C = 96
_S = 224
_NW = 32


def _sdesc(v):
    return plsc.sort_key_val(v, v, descending=True)[0]


def _rev(v):
    return lax.rev(v, dimensions=(0,))


def _merge16(a, b):
    rb = _rev(b)
    hi = jnp.maximum(a, rb)
    lo = jnp.minimum(a, rb)
    return _sdesc(hi), _sdesc(lo)


def _merge32(a, b):
    a0, a1 = a
    b0, b1 = b
    rb0 = _rev(b1)
    rb1 = _rev(b0)
    h0 = jnp.maximum(a0, rb0)
    h1 = jnp.maximum(a1, rb1)
    l0 = jnp.minimum(a0, rb0)
    l1 = jnp.minimum(a1, rb1)
    u0 = jnp.maximum(h0, h1)
    u1 = jnp.minimum(h0, h1)
    w0 = jnp.maximum(l0, l1)
    w1 = jnp.minimum(l0, l1)
    return _sdesc(u0), _sdesc(u1), _sdesc(w0), _sdesc(w1)


def _select47(q, p):
    q0, q1, q2, q3 = q
    p0, p1 = p
    h2 = jnp.maximum(q2, _rev(p1))
    h3 = jnp.maximum(q3, _rev(p0))
    e0 = jnp.minimum(q0, h2)
    e1 = jnp.minimum(q1, h3)
    f = jnp.maximum(e0, e1)
    return jnp.min(f)


def _chunk_compute(in_v, out_v):
    @plsc.parallel_loop(0, _S, unroll=4)
    def loc_body(s):
        sp = jnp.full((16,), s, jnp.int32)
        ys = []
        for j in range(6):
            cvec = lax.iota(jnp.int32, 16) + 16 * j
            v = plsc.load_gather(in_v, [cvec, sp])
            ys.append(jnp.maximum(v, 0.0))
        ss = [_sdesc(y) for y in ys]
        p1 = _merge16(ss[0], ss[1])
        p2 = _merge16(ss[2], ss[3])
        p3 = _merge16(ss[4], ss[5])
        q = _merge32(p1, p2)
        t = _select47(q, p3)
        for j in range(6):
            cvec = lax.iota(jnp.int32, 16) + 16 * j
            o = jnp.where(ys[j] >= t, ys[j], 0.0)
            plsc.store_scatter(out_v, [cvec, sp], o)


def kernel(x):
    B, C, H, W = x.shape
    assert C == _C
    HW = H * W
    RPB = HW // _S
    assert HW % _S == 0
    total_chunks = B * RPB
    CPW = total_chunks // _NW
    assert total_chunks % _NW == 0 and CPW % 2 == 0
    xv = x.reshape(B * C * RPB, _S)

    mesh = plsc.VectorSubcoreMesh(
        core_axis_name="c", subcore_axis_name="s", num_cores=2, num_subcores=16
    )

    @functools.partial(
        pl.kernel,
        out_type=jax.ShapeDtypeStruct((B * C * RPB, _S), jnp.float32),
        mesh=mesh,
        scratch_types=[
            pltpu.VMEM((C,), jnp.int32),
            pltpu.VMEM((C,), jnp.int32),
            pltpu.VMEM((C,), jnp.int32),
            pltpu.VMEM((C,), jnp.int32),
            pltpu.VMEM((C, _S), jnp.float32),
            pltpu.VMEM((C, _S), jnp.float32),
            pltpu.VMEM((C, _S), jnp.float32),
            pltpu.VMEM((C, _S), jnp.float32),
            pltpu.SemaphoreType.DMA,
            pltpu.SemaphoreType.DMA,
            pltpu.SemaphoreType.DMA,
            pltpu.SemaphoreType.DMA,
        ],
        compiler_params=pltpu.CompilerParams(
            use_tc_tiling_on_sc=False, needs_layout_passes=False
        ),
    )
    def run(x_hbm, o_hbm, gidx0, gidx1, sidx0, sidx1, in0, in1, out0, out1,
            gsem0, gsem1, ssem0, ssem1):
        wid = lax.axis_index("s") * 2 + lax.axis_index("c")
        base = wid * CPW
        gidx = (gidx0, gidx1)
        sidx = (sidx0, sidx1)
        inb = (in0, in1)
        outb = (out0, out1)
        gsem = (gsem0, gsem1)
        ssem = (ssem0, ssem1)

        def write_idx(ref, g):
            b = g // RPB
            m = g % RPB
            for j in range(6):
                cvec = lax.iota(jnp.int32, 16) + 16 * j
                ref[pl.ds(16 * j, 16)] = (b * C + cvec) * RPB + m

        write_idx(gidx0, base)
        pltpu.async_copy(x_hbm.at[gidx0], in0, gsem0)

        def outer(kk, carry):
            for p in (0, 1):
                q = 1 - p
                k = kk * 2 + p
                g = base + k

                @pl.when(k + 1 < CPW)
                def _():
                    write_idx(gidx[q], g + 1)
                    pltpu.async_copy(x_hbm.at[gidx[q]], inb[q], gsem[q])

                pltpu.make_async_copy(x_hbm.at[gidx[p]], inb[p], gsem[p]).wait()

                @pl.when(k >= 2)
                def _():
                    pltpu.make_async_copy(
                        outb[p], o_hbm.at[sidx[p]], ssem[p]
                    ).wait()

                _chunk_compute(inb[p], outb[p])
                write_idx(sidx[p], g)
                pltpu.async_copy(outb[p], o_hbm.at[sidx[p]], ssem[p])
            return carry

        lax.fori_loop(0, CPW // 2, outer, 0)
        pltpu.make_async_copy(outb[0], o_hbm.at[sidx[0]], ssem[0]).wait()
        pltpu.make_async_copy(outb[1], o_hbm.at[sidx[1]], ssem[1]).wait()

    out = run(xv)
    return out.reshape(B, C, H, W)


# speedup vs baseline: 1.1927x; 1.0007x over previous
import functools
import jax
import jax.numpy as jnp
from jax import lax
from jax.experimental import pallas as pl
from jax.experimental.pallas import tpu as pltpu
from jax.experimental.pallas import tpu_sc as plsc

_C = 96
_S = 224
_NW = 32


def _sasc(v):
    return lax.sort(v, dimension=0, is_stable=False, num_keys=1)


def _rev(v):
    return lax.rev(v, dimensions=(0,))


def _merge16(a, b):
    rb = _rev(b)
    lo = jnp.minimum(a, rb)
    hi = jnp.maximum(a, rb)
    return _sasc(lo), _sasc(hi)


def _merge32(a, b):
    a0, a1 = a
    b0, b1 = b
    rb0 = _rev(b1)
    rb1 = _rev(b0)
    l0 = jnp.minimum(a0, rb0)
    l1 = jnp.minimum(a1, rb1)
    h0 = jnp.maximum(a0, rb0)
    h1 = jnp.maximum(a1, rb1)
    u0 = jnp.minimum(l0, l1)
    u1 = jnp.maximum(l0, l1)
    w0 = jnp.minimum(h0, h1)
    w1 = jnp.maximum(h0, h1)
    return _sasc(u0), _sasc(u1), _sasc(w0), _sasc(w1)


def _select47(q, p):
    q0, q1, q2, q3 = q
    p0, p1 = p
    h2 = jnp.minimum(q2, _rev(p1))
    h3 = jnp.minimum(q3, _rev(p0))
    e0 = jnp.maximum(q0, h2)
    e1 = jnp.maximum(q1, h3)
    g = jnp.maximum(e0, e1)
    return jnp.min(g)


def _chunk_compute(in_v, out_v):
    @plsc.parallel_loop(0, _S, unroll=4)
    def loc_body(s):
        sp = jnp.full((16,), s, jnp.int32)
        ys = []
        for j in range(6):
            cvec = lax.iota(jnp.int32, 16) + 16 * j
            v = plsc.load_gather(in_v, [cvec, sp])
            ys.append(jnp.maximum(v, 0.0))
        ss = [_sasc(y) for y in ys]
        p1 = _merge16(ss[0], ss[1])
        p2 = _merge16(ss[2], ss[3])
        p3 = _merge16(ss[4], ss[5])
        q = _merge32(p1, p2)
        t = _select47(q, p3)
        for j in range(6):
            cvec = lax.iota(jnp.int32, 16) + 16 * j
            o = jnp.where(ys[j] >= t, ys[j], 0.0)
            plsc.store_scatter(out_v, [cvec, sp], o)


def kernel(x):
    B, C, H, W = x.shape
    assert C == _C
    HW = H * W
    RPB = HW // _S
    assert HW % _S == 0
    total_chunks = B * RPB
    CPW = total_chunks // _NW
    assert total_chunks % _NW == 0 and CPW % 2 == 0
    xv = x.reshape(B * C * RPB, _S)

    mesh = plsc.VectorSubcoreMesh(
        core_axis_name="c", subcore_axis_name="s", num_cores=2, num_subcores=16
    )

    @functools.partial(
        pl.kernel,
        out_type=jax.ShapeDtypeStruct((B * C * RPB, _S), jnp.float32),
        mesh=mesh,
        scratch_types=[
            pltpu.VMEM((C,), jnp.int32),
            pltpu.VMEM((C,), jnp.int32),
            pltpu.VMEM((C,), jnp.int32),
            pltpu.VMEM((C,), jnp.int32),
            pltpu.VMEM((C, _S), jnp.float32),
            pltpu.VMEM((C, _S), jnp.float32),
            pltpu.VMEM((C, _S), jnp.float32),
            pltpu.VMEM((C, _S), jnp.float32),
            pltpu.SemaphoreType.DMA,
            pltpu.SemaphoreType.DMA,
            pltpu.SemaphoreType.DMA,
            pltpu.SemaphoreType.DMA,
        ],
        compiler_params=pltpu.CompilerParams(
            use_tc_tiling_on_sc=False, needs_layout_passes=False
        ),
    )
    def run(x_hbm, o_hbm, gidx0, gidx1, sidx0, sidx1, in0, in1, out0, out1,
            gsem0, gsem1, ssem0, ssem1):
        wid = lax.axis_index("s") * 2 + lax.axis_index("c")
        base = wid * CPW
        gidx = (gidx0, gidx1)
        sidx = (sidx0, sidx1)
        inb = (in0, in1)
        outb = (out0, out1)
        gsem = (gsem0, gsem1)
        ssem = (ssem0, ssem1)

        def write_idx(ref, g):
            b = g // RPB
            m = g % RPB
            for j in range(6):
                cvec = lax.iota(jnp.int32, 16) + 16 * j
                ref[pl.ds(16 * j, 16)] = (b * C + cvec) * RPB + m

        write_idx(gidx0, base)
        pltpu.async_copy(x_hbm.at[gidx0], in0, gsem0)

        def outer(kk, carry):
            for p in (0, 1):
                q = 1 - p
                k = kk * 2 + p
                g = base + k

                @pl.when(k + 1 < CPW)
                def _():
                    write_idx(gidx[q], g + 1)
                    pltpu.async_copy(x_hbm.at[gidx[q]], inb[q], gsem[q])

                pltpu.make_async_copy(x_hbm.at[gidx[p]], inb[p], gsem[p]).wait()

                @pl.when(k >= 2)
                def _():
                    pltpu.make_async_copy(
                        outb[p], o_hbm.at[sidx[p]], ssem[p]
                    ).wait()

                _chunk_compute(inb[p], outb[p])
                write_idx(sidx[p], g)
                pltpu.async_copy(outb[p], o_hbm.at[sidx[p]], ssem[p])
            return carry

        lax.fori_loop(0, CPW // 2, outer, 0)
        pltpu.make_async_copy(outb[0], o_hbm.at[sidx[0]], ssem[0]).wait()
        pltpu.make_async_copy(outb[1], o_hbm.at[sidx[1]], ssem[1]).wait()

    out = run(xv)
    return out.reshape(B, C, H, W)


# speedup vs baseline: 2.3545x; 1.9741x over previous
import functools
import jax
import jax.numpy as jnp
from jax import lax
from jax.experimental import pallas as pl
from jax.experimental.pallas import tpu as pltpu
from jax.experimental.pallas import tpu_sc as plsc

_C = 96
_S = 224
_P = 232
_NW = 32


def _sasc(v):
    return lax.sort(v, dimension=0, is_stable=False, num_keys=1)


def _rev(v):
    return lax.rev(v, dimensions=(0,))


def _merge16(a, b):
    rb = _rev(b)
    lo = jnp.minimum(a, rb)
    hi = jnp.maximum(a, rb)
    return _sasc(lo), _sasc(hi)


def _merge32(a, b):
    a0, a1 = a
    b0, b1 = b
    rb0 = _rev(b1)
    rb1 = _rev(b0)
    l0 = jnp.minimum(a0, rb0)
    l1 = jnp.minimum(a1, rb1)
    h0 = jnp.maximum(a0, rb0)
    h1 = jnp.maximum(a1, rb1)
    u0 = jnp.minimum(l0, l1)
    u1 = jnp.maximum(l0, l1)
    w0 = jnp.minimum(h0, h1)
    w1 = jnp.maximum(h0, h1)
    return _sasc(u0), _sasc(u1), _sasc(w0), _sasc(w1)


def _select47(q, p):
    q0, q1, q2, q3 = q
    p0, p1 = p
    h2 = jnp.minimum(q2, _rev(p1))
    h3 = jnp.minimum(q3, _rev(p0))
    e0 = jnp.maximum(q0, h2)
    e1 = jnp.maximum(q1, h3)
    g = jnp.maximum(e0, e1)
    return jnp.min(g)


def _chunk_compute(in_v, out_v):
    @plsc.parallel_loop(0, _S, unroll=4)
    def loc_body(s):
        sp = jnp.full((16,), s, jnp.int32)
        ys = []
        for j in range(6):
            cvec = lax.iota(jnp.int32, 16) + 16 * j
            v = plsc.load_gather(in_v, [cvec, sp])
            ys.append(jnp.maximum(v, 0.0))
        ss = [_sasc(y) for y in ys]
        p1 = _merge16(ss[0], ss[1])
        p2 = _merge16(ss[2], ss[3])
        p3 = _merge16(ss[4], ss[5])
        q = _merge32(p1, p2)
        t = _select47(q, p3)
        for j in range(6):
            cvec = lax.iota(jnp.int32, 16) + 16 * j
            o = jnp.where(ys[j] >= t, ys[j], 0.0)
            plsc.store_scatter(out_v, [cvec, sp], o)


def kernel(x):
    B, C, H, W = x.shape
    assert C == _C
    HW = H * W
    RPB = HW // _S
    assert HW % _S == 0
    total_chunks = B * RPB
    CPW = total_chunks // _NW
    assert total_chunks % _NW == 0 and CPW % 2 == 0
    xv = x.reshape(B * C * RPB, _S)

    mesh = plsc.VectorSubcoreMesh(
        core_axis_name="c", subcore_axis_name="s", num_cores=2, num_subcores=16
    )

    @functools.partial(
        pl.kernel,
        out_type=jax.ShapeDtypeStruct((B * C * RPB, _S), jnp.float32),
        mesh=mesh,
        scratch_types=[
            pltpu.VMEM((C, _P), jnp.float32),
            pltpu.VMEM((C, _P), jnp.float32),
            pltpu.VMEM((C, _P), jnp.float32),
            pltpu.VMEM((C, _P), jnp.float32),
            pltpu.SemaphoreType.DMA,
            pltpu.SemaphoreType.DMA,
            pltpu.SemaphoreType.DMA,
            pltpu.SemaphoreType.DMA,
        ],
        compiler_params=pltpu.CompilerParams(
            use_tc_tiling_on_sc=False, needs_layout_passes=False
        ),
    )
    def run(x_hbm, o_hbm, in0, in1, out0, out1, gsem0, gsem1, ssem0, ssem1):
        wid = lax.axis_index("s") * 2 + lax.axis_index("c")
        base = wid * CPW
        inb = (in0, in1)
        outb = (out0, out1)
        gsem = (gsem0, gsem1)
        ssem = (ssem0, ssem1)

        def issue_gather(g, ref, sem):
            b = g // RPB
            m = g % RPB
            row0 = b * C * RPB + m
            for c in range(C):
                pltpu.async_copy(
                    x_hbm.at[row0 + c * RPB], ref.at[c, pl.ds(0, _S)], sem
                )

        def drain_gather(ref, sem):
            for c in range(C):
                pltpu.make_async_copy(
                    x_hbm.at[0], ref.at[c, pl.ds(0, _S)], sem
                ).wait()

        def issue_scatter(g, ref, sem):
            b = g // RPB
            m = g % RPB
            row0 = b * C * RPB + m
            for c in range(C):
                pltpu.async_copy(
                    ref.at[c, pl.ds(0, _S)], o_hbm.at[row0 + c * RPB], sem
                )

        def drain_scatter(ref, sem):
            for c in range(C):
                pltpu.make_async_copy(
                    ref.at[c, pl.ds(0, _S)], o_hbm.at[0], sem
                ).wait()

        issue_gather(base, in0, gsem0)

        def outer(kk, carry):
            for p in (0, 1):
                q = 1 - p
                k = kk * 2 + p
                g = base + k

                @pl.when(k + 1 < CPW)
                def _():
                    issue_gather(g + 1, inb[q], gsem[q])

                drain_gather(inb[p], gsem[p])

                @pl.when(k >= 2)
                def _():
                    drain_scatter(outb[p], ssem[p])

                _chunk_compute(inb[p], outb[p])
                issue_scatter(g, outb[p], ssem[p])
            return carry

        lax.fori_loop(0, CPW // 2, outer, 0)
        drain_scatter(outb[0], ssem[0])
        drain_scatter(outb[1], ssem[1])

    out = run(xv)
    return out.reshape(B, C, H, W)
